# Initial kernel scaffold; baseline (speedup 1.0000x reference)
import jax
import jax.numpy as jnp
from jax.experimental import pallas as pl


def kernel(x, table):
    raise NotImplementedError("write your pallas kernel here")


# speedup vs baseline: 4.9499x; 4.9499x over previous
import functools

import jax
import jax.numpy as jnp
from jax import lax
from jax.experimental import pallas as pl
from jax.experimental.pallas import tpu as pltpu
from jax.experimental.pallas import tpu_sc as plsc

DIM = 32
CHUNK = 2048


@functools.cache
def _build(n_total, dim):
    info = plsc.get_sparse_core_info()
    nw = info.num_cores * info.num_subcores
    per_w = n_total // nw
    assert per_w * nw == n_total
    n_ch = per_w // CHUNK
    assert n_ch * CHUNK == per_w

    mesh = plsc.VectorSubcoreMesh(core_axis_name="c", subcore_axis_name="s")

    @functools.partial(
        pl.kernel,
        mesh=mesh,
        out_type=jax.ShapeDtypeStruct((n_total, dim), jnp.float32),
        scratch_types=[
            pltpu.VMEM((CHUNK,), jnp.int32),
            pltpu.VMEM((CHUNK, dim), jnp.float32),
            pltpu.SemaphoreType.DMA,
        ],
        compiler_params=pltpu.CompilerParams(use_tc_tiling_on_sc=False),
    )
    def k(idx_hbm, table_hbm, out_hbm, idx_v, rows_v, sem):
        wid = lax.axis_index("s") * info.num_cores + lax.axis_index("c")
        base = wid * per_w

        def body(i, _):
            off = base + i * CHUNK
            pltpu.sync_copy(idx_hbm.at[pl.ds(off, CHUNK)], idx_v)
            pltpu.async_copy(table_hbm.at[idx_v], rows_v, sem).wait()
            pltpu.sync_copy(rows_v, out_hbm.at[pl.ds(off, CHUNK)])
            return 0

        lax.fori_loop(0, n_ch, body, 0)

    return k


def kernel(x, table):
    b, l = x.shape
    flat = x.reshape(b * l)
    out = _build(b * l, table.shape[1])(flat, table)
    return out.reshape(b, l, table.shape[1])


# speedup vs baseline: 6.8882x; 1.3916x over previous
import functools

import jax
import jax.numpy as jnp
from jax import lax
from jax.experimental import pallas as pl
from jax.experimental.pallas import tpu as pltpu
from jax.experimental.pallas import tpu_sc as plsc

DIM = 32
CHUNK = 1600


@functools.cache
def _build(n_total, dim):
    info = plsc.get_sparse_core_info()
    nw = info.num_cores * info.num_subcores
    per_w = n_total // nw
    assert per_w * nw == n_total
    n_ch = per_w // CHUNK
    assert n_ch * CHUNK == per_w and n_ch % 2 == 0

    mesh = plsc.VectorSubcoreMesh(core_axis_name="c", subcore_axis_name="s")

    @functools.partial(
        pl.kernel,
        mesh=mesh,
        out_type=jax.ShapeDtypeStruct((n_total, dim), jnp.float32),
        scratch_types=[
            pltpu.VMEM((2, CHUNK), jnp.int32),
            pltpu.VMEM((2, CHUNK, dim), jnp.float32),
            pltpu.SemaphoreType.DMA((2,)),
            pltpu.SemaphoreType.DMA((2,)),
        ],
        compiler_params=pltpu.CompilerParams(use_tc_tiling_on_sc=False),
    )
    def k(idx_hbm, table_hbm, out_hbm, idx_v, rows_v, gsem, osem):
        wid = lax.axis_index("s") * info.num_cores + lax.axis_index("c")
        base = wid * per_w

        def idx_chunk(i):
            return idx_hbm.at[pl.ds(base + i * CHUNK, CHUNK)]

        def out_chunk(i):
            return out_hbm.at[pl.ds(base + i * CHUNK, CHUNK)]

        pltpu.sync_copy(idx_chunk(0), idx_v.at[0])
        pltpu.async_copy(table_hbm.at[idx_v.at[0]], rows_v.at[0], gsem.at[0])

        def outer(j, _):
            g = j * 2
            for b in range(2):
                i = g + b
                nb = 1 - b

                @pl.when(i + 1 < n_ch)
                def _stage_next():
                    pltpu.sync_copy(idx_chunk(i + 1), idx_v.at[nb])

                    @pl.when(i >= 1)
                    def _drain_out():
                        pltpu.make_async_copy(
                            rows_v.at[nb], out_chunk(i - 1), osem.at[nb]
                        ).wait()

                    pltpu.async_copy(
                        table_hbm.at[idx_v.at[nb]], rows_v.at[nb], gsem.at[nb]
                    )

                pltpu.make_async_copy(
                    table_hbm.at[idx_v.at[b]], rows_v.at[b], gsem.at[b]
                ).wait()
                pltpu.async_copy(rows_v.at[b], out_chunk(i), osem.at[b])
            return 0

        lax.fori_loop(0, n_ch // 2, outer, 0)

        pltpu.make_async_copy(rows_v.at[0], out_chunk(0), osem.at[0]).wait()
        pltpu.make_async_copy(rows_v.at[1], out_chunk(1), osem.at[1]).wait()

    return k


def kernel(x, table):
    b, l = x.shape
    flat = x.reshape(b * l)
    out = _build(b * l, table.shape[1])(flat, table)
    return out.reshape(b, l, table.shape[1])
